# baseline (speedup 1.0000x reference)
import functools

import jax
import jax.numpy as jnp
from jax import lax
from jax.experimental import pallas as pl
from jax.experimental.pallas import tpu as pltpu
from jax.experimental.pallas import tpu_sc as plsc

VOCAB = 100000
D_MODEL = 2048
SEQ = 8192

_NUM_CORES = 2
_NUM_SUBCORES = 16
_NW = _NUM_CORES * _NUM_SUBCORES
_PER_W = SEQ // _NW
_CHUNK = 8
_NCHUNK = _PER_W // _CHUNK
_NBUF = 6


def _gather_kernel(table_hbm, idx_hbm, out_hbm, idx_v, *scratch):
    bufs = scratch[:_NBUF]
    gsems = scratch[_NBUF:2 * _NBUF]
    osems = scratch[2 * _NBUF:3 * _NBUF]

    wid = lax.axis_index("s") * _NUM_CORES + lax.axis_index("c")
    base = wid * _PER_W

    pltpu.sync_copy(idx_hbm.at[wid], idx_v)

    gathers = [None] * _NBUF
    outcopies = [None] * _NBUF

    for j in range(min(_NBUF - 1, _NCHUNK)):
        gathers[j] = pltpu.async_copy(
            table_hbm.at[idx_v.at[j]], bufs[j], gsems[j])

    for j in range(_NCHUNK):
        b = j % _NBUF
        nxt = j + _NBUF - 1
        if nxt < _NCHUNK:
            nb = nxt % _NBUF
            if outcopies[nb] is not None:
                outcopies[nb].wait()
            gathers[nb] = pltpu.async_copy(
                table_hbm.at[idx_v.at[nxt]], bufs[nb], gsems[nb])
        gathers[b].wait()
        outcopies[b] = pltpu.async_copy(
            bufs[b], out_hbm.at[pl.ds(base + j * _CHUNK, _CHUNK)], osems[b])

    for b in range(_NBUF):
        if outcopies[b] is not None:
            outcopies[b].wait()


@jax.jit
def _embedding_gather(idx, table):
    mesh = plsc.VectorSubcoreMesh(core_axis_name="c", subcore_axis_name="s")
    scratch = (
        [pltpu.VMEM((_NCHUNK, _CHUNK), jnp.int32)]
        + [pltpu.VMEM((_CHUNK, D_MODEL), jnp.float32)] * _NBUF
        + [pltpu.SemaphoreType.DMA] * (2 * _NBUF)
    )
    k = functools.partial(
        pl.kernel,
        mesh=mesh,
        out_type=jax.ShapeDtypeStruct((SEQ, D_MODEL), jnp.float32),
        scratch_types=scratch,
    )(_gather_kernel)
    return k(table, idx)


def kernel(x, embd_weights):
    idx = jnp.reshape(x.astype(jnp.int32), (_NW, _NCHUNK, _CHUNK))
    out = _embedding_gather(idx, embd_weights)
    return jnp.reshape(out, (1, 1, SEQ, D_MODEL))


# speedup vs baseline: 1.0006x; 1.0006x over previous
import functools

import jax
import jax.numpy as jnp
from jax import lax
from jax.experimental import pallas as pl
from jax.experimental.pallas import tpu as pltpu
from jax.experimental.pallas import tpu_sc as plsc

VOCAB = 100000
D_MODEL = 2048
SEQ = 8192

_NUM_CORES = 2
_NUM_SUBCORES = 16
_NW = _NUM_CORES * _NUM_SUBCORES
_PER_W = SEQ // _NW
_CHUNK = 16
_NCHUNK = _PER_W // _CHUNK
_NBUF = 3


def _gather_kernel(table_hbm, idx_hbm, out_hbm, idx_v, *scratch):
    bufs = scratch[:_NBUF]
    gsems = scratch[_NBUF:2 * _NBUF]
    osems = scratch[2 * _NBUF:3 * _NBUF]

    wid = lax.axis_index("s") * _NUM_CORES + lax.axis_index("c")
    base = wid * _PER_W

    pltpu.sync_copy(idx_hbm.at[0, pl.ds(base, _PER_W)], idx_v)

    gathers = [None] * _NBUF
    outcopies = [None] * _NBUF

    for j in range(min(_NBUF - 1, _NCHUNK)):
        gathers[j] = pltpu.async_copy(
            table_hbm.at[idx_v.at[pl.ds(j * _CHUNK, _CHUNK)]],
            bufs[j], gsems[j])

    for j in range(_NCHUNK):
        b = j % _NBUF
        nxt = j + _NBUF - 1
        if nxt < _NCHUNK:
            nb = nxt % _NBUF
            if outcopies[nb] is not None:
                outcopies[nb].wait()
            gathers[nb] = pltpu.async_copy(
                table_hbm.at[idx_v.at[pl.ds(nxt * _CHUNK, _CHUNK)]],
                bufs[nb], gsems[nb])
        gathers[b].wait()
        outcopies[b] = pltpu.async_copy(
            bufs[b],
            out_hbm.at[0, 0, pl.ds(base + j * _CHUNK, _CHUNK)],
            osems[b])

    for b in range(_NBUF):
        if outcopies[b] is not None:
            outcopies[b].wait()


@jax.jit
def _embedding_gather(idx, table):
    mesh = plsc.VectorSubcoreMesh(core_axis_name="c", subcore_axis_name="s")
    scratch = (
        [pltpu.VMEM((_PER_W,), jnp.int32)]
        + [pltpu.VMEM((_CHUNK, D_MODEL), jnp.float32)] * _NBUF
        + [pltpu.SemaphoreType.DMA] * (2 * _NBUF)
    )
    k = functools.partial(
        pl.kernel,
        mesh=mesh,
        out_type=jax.ShapeDtypeStruct((1, 1, SEQ, D_MODEL), jnp.float32),
        scratch_types=scratch,
    )(_gather_kernel)
    return k(table, idx)


def kernel(x, embd_weights):
    return _embedding_gather(x.astype(jnp.int32), embd_weights)
